# baseline (speedup 1.0000x reference)
import jax
import jax.numpy as jnp
from jax.experimental import pallas as pl
from jax.experimental.pallas import tpu as pltpu

EMB = 64
NR = 256
NO = 1024
NS = NR + NO
TR = 16
GRID = NR // TR


def _lrelu(x):
    return jnp.maximum(x, 0.01 * x)


_relu = jax.nn.relu


def _prep_kernel(x_all_ref, x_allT_ref,
                 We1, be1, We2, be2,
                 We1T, be1c, We2T, be2c,
                 Wo1T, bo1c, Wo2T, bo2c,
                 Wq, Wa1q, ba1,
                 Wk_r, Wv_r, bv, Whr_rec, Who_rec,
                 Wk_sT, Wv_sT,
                 Whard1h, Whr_xT, bhrc,
                 Whardo1h, bhoc,
                 hr_o, A1q_o, Ki_o, Vi_o, Rgr_o, Rgo_o,
                 KjT2_o, VjT2_o, SgrT2_o, SgoT2_o):
    x_all = x_all_ref[:, :]
    x_r = x_all[:NR]
    x_allT = x_allT_ref[:, :]
    x_rT = x_allT[:, :NR]
    x_oT = x_allT[:, NR:]

    h_r = _lrelu(jnp.dot(_lrelu(jnp.dot(x_r, We1[:, :]) + be1[:, :]),
                         We2[:, :]) + be2[:, :])
    hr_o[:, :] = h_r
    q = jnp.dot(h_r, Wq[:, :])
    A1q_o[:, :] = jnp.dot(q, Wa1q[:, :]) + ba1[:, :]
    Ki_o[:, :] = jnp.dot(x_r, Wk_r[:, :])
    Vi_o[:, :] = jnp.dot(x_r, Wv_r[:, :]) + bv[:, :]
    Rgr_o[:, :] = jnp.dot(x_r, Whr_rec[:, :])
    Rgo_o[:, :] = jnp.dot(x_r, Who_rec[:, :])

    h_rT = _lrelu(jnp.dot(We2T[:, :],
                          _lrelu(jnp.dot(We1T[:, :], x_rT) + be1c[:, :]))
                  + be2c[:, :])
    h_oT = _lrelu(jnp.dot(Wo2T[:, :],
                          _lrelu(jnp.dot(Wo1T[:, :], x_oT) + bo1c[:, :]))
                  + bo2c[:, :])
    kjT = jnp.dot(Wk_sT[:, :], x_allT)
    KjT2_o[:, :] = jnp.concatenate([kjT, kjT], axis=0)
    vjT = jnp.dot(Wv_sT[:, :], x_allT)
    VjT2_o[:, :] = jnp.concatenate([vjT, vjT], axis=0)
    sgrT = (jnp.dot(Whard1h[:, :], h_rT)
            + jnp.dot(Whr_xT[:, :], x_rT) + bhrc[:, :])
    SgrT2_o[:, :] = jnp.concatenate([sgrT, sgrT], axis=0)
    sgoT = jnp.dot(Whardo1h[:, :], h_oT) + bhoc[:, :]
    SgoT2_o[:, :] = jnp.concatenate([sgoT, sgoT], axis=0)


def _main_kernel(hr_r, A1q_r, Ki_r, Vi_r, Rgr_r, Rgo_r,
                 KjT2_r, VjT2_r, SgrT2_r, SgoT2_r,
                 A1k_bd, a2_2, wrr2, crr, wor2, cor,
                 Wd1, bd1c, Wd2, bd2c,
                 o_ref):
    hrT_t = jnp.transpose(hr_r[:, :])
    A1qT_t = jnp.transpose(A1q_r[:, :])
    KiT_t = jnp.transpose(Ki_r[:, :])
    ViT_t = jnp.transpose(Vi_r[:, :])
    RgrT_t = jnp.transpose(Rgr_r[:, :])
    RgoT_t = jnp.transpose(Rgo_r[:, :])

    KjT2 = KjT2_r[:, :]
    VjT2 = VjT2_r[:, :]
    SgrT2 = SgrT2_r[:, :]
    SgoT2 = SgoT2_r[:, :]

    def cols2(a, s):
        return jnp.concatenate(
            [a[:, 2 * s:2 * s + 1], a[:, 2 * s + 1:2 * s + 2]], axis=0)

    outs = []
    for s in range(TR // 2):
        ke2 = _lrelu(cols2(KiT_t, s) + KjT2)
        pre2 = _relu(jnp.dot(A1k_bd[:, :], ke2,
                             preferred_element_type=jnp.float32)
                     + cols2(A1qT_t, s))
        sc2 = jnp.dot(a2_2[:, :], pre2,
                      preferred_element_type=jnp.float32)

        pgr2 = _relu(SgrT2 + cols2(RgrT_t, s))
        gr2 = jnp.dot(wrr2[:, :], pgr2,
                      preferred_element_type=jnp.float32) + crr[:, :]
        pgo2 = _relu(SgoT2 + cols2(RgoT_t, s))
        go2 = jnp.dot(wor2[:, :], pgo2,
                      preferred_element_type=jnp.float32) + cor[:, :]
        gate2 = jax.nn.sigmoid(jnp.concatenate([gr2, go2], axis=1))

        m2 = jnp.max(sc2, axis=1, keepdims=True)
        ex2 = jnp.exp(sc2 - m2)
        den2 = jnp.sum(ex2, axis=1, keepdims=True)
        u2 = ex2 * gate2

        ve2 = _lrelu(cols2(ViT_t, s) + VjT2)
        o_a = jnp.sum(ve2[:EMB] * u2[0:1], axis=1, keepdims=True)
        o_b = jnp.sum(ve2[EMB:] * u2[1:2], axis=1, keepdims=True)
        outs.append(o_a / (den2[0:1] + 1e-16))
        outs.append(o_b / (den2[1:2] + 1e-16))

    outT = jnp.concatenate(outs, axis=1)
    dec_inT = jnp.concatenate([hrT_t, outT], axis=0)
    yT = jnp.dot(Wd2[:, :],
                 _lrelu(jnp.dot(Wd1[:, :], dec_inT) + bd1c[:, :])) + bd2c[:, :]
    o_ref[0, :, :] = yT


def _pad_rows(a, rows):
    return jnp.zeros((rows, a.shape[1]), a.dtype).at[:a.shape[0]].set(a)


def _pad_cols(a, cols):
    return jnp.zeros((a.shape[0], cols), a.dtype).at[:, :a.shape[1]].set(a)


def _rep_spec(a):
    nd = a.ndim
    return pl.BlockSpec(a.shape, lambda *_, _nd=nd: (0,) * _nd)


def kernel(robot_embedding, obstacle_embedding, params):
    p = params
    f32 = jnp.float32

    x_all = jnp.concatenate([robot_embedding, obstacle_embedding], axis=0)
    x_all = jnp.concatenate(
        [x_all, jnp.zeros((NS, 3), f32)], axis=1)
    x_allT = x_all.T

    We1 = _pad_rows(p['emb1_W'].T, 8)
    be1 = p['emb1_b'].reshape(1, -1)
    We2 = p['emb2_W'].T
    be2 = p['emb2_b'].reshape(1, -1)
    We1T = _pad_cols(p['emb1_W'], 8)
    be1c = p['emb1_b'].reshape(-1, 1)
    We2T = p['emb2_W']
    be2c = p['emb2_b'].reshape(-1, 1)
    Wo1T = _pad_cols(p['oemb1_W'], 8)
    bo1c = p['oemb1_b'].reshape(-1, 1)
    Wo2T = p['oemb2_W']
    bo2c = p['oemb2_b'].reshape(-1, 1)

    Wq = p['q_W'].T
    Wa1q = p['attn1_W'][:, :EMB].T
    ba1 = p['attn1_b'].reshape(1, -1)

    Wk_r = _pad_rows(p['k_W'][:, 5:].T, 8)
    Wv_r = _pad_rows(p['v_W'][:, 5:].T, 8)
    bv = p['v_b'].reshape(1, -1)
    Whr_rec = _pad_rows(p['hard1_W'][:, EMB:EMB + 5].T, 8)
    Who_rec = _pad_rows(p['hardo1_W'][:, EMB:EMB + 5].T, 8)

    Wk_sT = _pad_cols(p['k_W'][:, :5], 8)
    Wv_sT = _pad_cols(p['v_W'][:, :5], 8)
    Whard1h = p['hard1_W'][:, :EMB]
    Whr_xT = jnp.zeros((EMB, 8), f32).at[:, 3:5].set(p['hard1_W'][:, EMB + 5:])
    bhrc = p['hard1_b'].reshape(-1, 1)
    Whardo1h = p['hardo1_W'][:, :EMB]
    bhoc = p['hardo1_b'].reshape(-1, 1)

    A1k = p['attn1_W'][:, EMB:]
    A1k_bd = jnp.zeros((2 * EMB, 2 * EMB), f32)
    A1k_bd = A1k_bd.at[:EMB, :EMB].set(A1k).at[EMB:, EMB:].set(A1k)
    a2row = p['attn2_W']
    a2_2 = jnp.zeros((2, 2 * EMB), f32)
    a2_2 = a2_2.at[0:1, :EMB].set(a2row).at[1:2, EMB:].set(a2row)

    d_r = p['henc_W'][1] - p['henc_W'][0]
    wrr = (p['hard2_W'].T @ d_r).reshape(1, EMB)
    crr = (p['hard2_b'] @ d_r + p['henc_b'][1] - p['henc_b'][0]).reshape(1, 1)
    d_o = p['henco_W'][1] - p['henco_W'][0]
    wor = (p['hardo2_W'].T @ d_o).reshape(1, EMB)
    cor = (p['hardo2_b'] @ d_o + p['henco_b'][1] - p['henco_b'][0]).reshape(1, 1)
    wrr2 = jnp.zeros((2, 2 * EMB), f32)
    wrr2 = wrr2.at[0:1, :EMB].set(wrr).at[1:2, EMB:].set(wrr)
    wor2 = jnp.zeros((2, 2 * EMB), f32)
    wor2 = wor2.at[0:1, :EMB].set(wor).at[1:2, EMB:].set(wor)

    Wd1 = p['dec1_W']
    bd1c = p['dec1_b'].reshape(-1, 1)
    Wd2 = p['dec2_W']
    bd2c = p['dec2_b'].reshape(-1, 1)

    prep_inputs = [x_all, x_allT,
                   We1, be1, We2, be2,
                   We1T, be1c, We2T, be2c,
                   Wo1T, bo1c, Wo2T, bo2c,
                   Wq, Wa1q, ba1,
                   Wk_r, Wv_r, bv, Whr_rec, Who_rec,
                   Wk_sT, Wv_sT,
                   Whard1h, Whr_xT, bhrc,
                   Whardo1h, bhoc]

    prep_outs = pl.pallas_call(
        _prep_kernel,
        in_specs=[_rep_spec(a) for a in prep_inputs],
        out_specs=[
            pl.BlockSpec((NR, EMB), lambda *_: (0, 0)),
            pl.BlockSpec((NR, EMB), lambda *_: (0, 0)),
            pl.BlockSpec((NR, EMB), lambda *_: (0, 0)),
            pl.BlockSpec((NR, EMB), lambda *_: (0, 0)),
            pl.BlockSpec((NR, EMB), lambda *_: (0, 0)),
            pl.BlockSpec((NR, EMB), lambda *_: (0, 0)),
            pl.BlockSpec((2 * EMB, NS), lambda *_: (0, 0)),
            pl.BlockSpec((2 * EMB, NS), lambda *_: (0, 0)),
            pl.BlockSpec((2 * EMB, NR), lambda *_: (0, 0)),
            pl.BlockSpec((2 * EMB, NO), lambda *_: (0, 0)),
        ],
        out_shape=[
            jax.ShapeDtypeStruct((NR, EMB), f32),
            jax.ShapeDtypeStruct((NR, EMB), f32),
            jax.ShapeDtypeStruct((NR, EMB), f32),
            jax.ShapeDtypeStruct((NR, EMB), f32),
            jax.ShapeDtypeStruct((NR, EMB), f32),
            jax.ShapeDtypeStruct((NR, EMB), f32),
            jax.ShapeDtypeStruct((2 * EMB, NS), f32),
            jax.ShapeDtypeStruct((2 * EMB, NS), f32),
            jax.ShapeDtypeStruct((2 * EMB, NR), f32),
            jax.ShapeDtypeStruct((2 * EMB, NO), f32),
        ],
    )(*prep_inputs)

    hr, A1q, Ki, Vi, Rgr, Rgo, KjT2, VjT2, SgrT2, SgoT2 = prep_outs

    def tile_spec(a):
        return pl.BlockSpec((TR, EMB), lambda i: (i, 0))

    main_inputs = [hr, A1q, Ki, Vi, Rgr, Rgo,
                   KjT2, VjT2, SgrT2, SgoT2,
                   A1k_bd, a2_2, wrr2, crr, wor2, cor,
                   Wd1, bd1c, Wd2, bd2c]
    main_specs = ([tile_spec(a) for a in main_inputs[:6]]
                  + [_rep_spec(a) for a in main_inputs[6:]])

    yT = pl.pallas_call(
        _main_kernel,
        grid=(GRID,),
        in_specs=main_specs,
        out_specs=pl.BlockSpec((1, 2 * EMB, TR), lambda i: (i, 0, 0)),
        out_shape=jax.ShapeDtypeStruct((GRID, 2 * EMB, TR), f32),
        compiler_params=pltpu.CompilerParams(
            dimension_semantics=("parallel",)),
    )(*main_inputs)
    return yT.transpose(0, 2, 1).reshape(NR, 2 * EMB)


# speedup vs baseline: 1.0723x; 1.0723x over previous
import jax
import jax.numpy as jnp
from jax.experimental import pallas as pl
from jax.experimental.pallas import tpu as pltpu

EMB = 64
NR = 256
NO = 1024
NS = NR + NO
TR = 16
GRID = NR // TR

def _lrelu(x):
    return jnp.maximum(x, 0.01 * x)


_relu = jax.nn.relu


def _fused(x_all_ref, x_allT_ref,
           We1, be1, We2, be2,
           We1T, be1c, We2T, be2c,
           Wo1T, bo1c, Wo2T, bo2c,
           Wq, Wa1q, ba1,
           Wk_r, Wv_r, bv, Whr_rec, Who_rec,
           Wk_sT, Wv_sT,
           Whard1h, Whr_xT, bhrc,
           Whardo1h, bhoc,
           A1k_bd, a2_2, wrr2, crr, wor2, cor,
           Wd1, bd1c, Wd2, bd2c,
           o_ref,
           hr_s, A1q_s, Ki_s, Vi_s, Rgr_s, Rgo_s,
           KjT2_s, VjT2_s,
           SgrT2_s, SgoT2_s):
    pid = pl.program_id(0)

    @pl.when(pid == 0)
    def _prep():
        x_all = x_all_ref[:, :]
        x_r = x_all[:NR]
        x_allT = x_allT_ref[:, :]
        x_rT = x_allT[:, :NR]
        x_oT = x_allT[:, NR:]

        h_r = _lrelu(jnp.dot(_lrelu(jnp.dot(x_r, We1[:, :]) + be1[:, :]),
                             We2[:, :]) + be2[:, :])
        hr_s[:, :] = h_r
        q = jnp.dot(h_r, Wq[:, :])
        A1q_s[:, :] = jnp.dot(q, Wa1q[:, :]) + ba1[:, :]
        Ki_s[:, :] = jnp.dot(x_r, Wk_r[:, :])
        Vi_s[:, :] = jnp.dot(x_r, Wv_r[:, :]) + bv[:, :]
        Rgr_s[:, :] = jnp.dot(x_r, Whr_rec[:, :])
        Rgo_s[:, :] = jnp.dot(x_r, Who_rec[:, :])

        h_rT = _lrelu(jnp.dot(We2T[:, :],
                              _lrelu(jnp.dot(We1T[:, :], x_rT) + be1c[:, :]))
                      + be2c[:, :])
        h_oT = _lrelu(jnp.dot(Wo2T[:, :],
                              _lrelu(jnp.dot(Wo1T[:, :], x_oT) + bo1c[:, :]))
                      + bo2c[:, :])
        kjT = jnp.dot(Wk_sT[:, :], x_allT)
        KjT2_s[:, :] = jnp.concatenate([kjT, kjT], axis=0)
        vjT = jnp.dot(Wv_sT[:, :], x_allT)
        VjT2_s[:, :] = jnp.concatenate([vjT, vjT], axis=0)
        sgrT = (jnp.dot(Whard1h[:, :], h_rT)
                + jnp.dot(Whr_xT[:, :], x_rT) + bhrc[:, :])
        SgrT2_s[:, :] = jnp.concatenate([sgrT, sgrT], axis=0)
        sgoT = jnp.dot(Whardo1h[:, :], h_oT) + bhoc[:, :]
        SgoT2_s[:, :] = jnp.concatenate([sgoT, sgoT], axis=0)

    i0 = pid * TR
    hrT_t = jnp.transpose(hr_s[pl.ds(i0, TR), :])
    A1qT_t = jnp.transpose(A1q_s[pl.ds(i0, TR), :])
    KiT_t = jnp.transpose(Ki_s[pl.ds(i0, TR), :])
    ViT_t = jnp.transpose(Vi_s[pl.ds(i0, TR), :])
    RgrT_t = jnp.transpose(Rgr_s[pl.ds(i0, TR), :])
    RgoT_t = jnp.transpose(Rgo_s[pl.ds(i0, TR), :])

    KjT2 = KjT2_s[:, :]
    VjT2 = VjT2_s[:, :]
    SgrT2 = SgrT2_s[:, :]
    SgoT2 = SgoT2_s[:, :]

    def cols2(a, s):
        return jnp.concatenate(
            [a[:, 2 * s:2 * s + 1], a[:, 2 * s + 1:2 * s + 2]], axis=0)

    outs = []
    for s in range(TR // 2):
        ke2 = _lrelu(cols2(KiT_t, s) + KjT2)
        pre2 = _relu(jnp.dot(A1k_bd[:, :], ke2,
                             preferred_element_type=jnp.float32)
                     + cols2(A1qT_t, s))
        sc2 = jnp.dot(a2_2[:, :], pre2,
                      preferred_element_type=jnp.float32)

        pgr2 = _relu(SgrT2 + cols2(RgrT_t, s))
        gr2 = jnp.dot(wrr2[:, :], pgr2,
                      preferred_element_type=jnp.float32) + crr[:, :]
        pgo2 = _relu(SgoT2 + cols2(RgoT_t, s))
        go2 = jnp.dot(wor2[:, :], pgo2,
                      preferred_element_type=jnp.float32) + cor[:, :]
        gate2 = jax.nn.sigmoid(jnp.concatenate([gr2, go2], axis=1))

        m2 = jnp.max(sc2, axis=1, keepdims=True)
        ex2 = jnp.exp(sc2 - m2)
        den2 = jnp.sum(ex2, axis=1, keepdims=True)
        u2 = ex2 * gate2

        ve2 = _lrelu(cols2(ViT_t, s) + VjT2)
        o_a = jnp.sum(ve2[:EMB] * u2[0:1], axis=1, keepdims=True)
        o_b = jnp.sum(ve2[EMB:] * u2[1:2], axis=1, keepdims=True)
        outs.append(o_a / (den2[0:1] + 1e-16))
        outs.append(o_b / (den2[1:2] + 1e-16))

    outT = jnp.concatenate(outs, axis=1)
    dec_inT = jnp.concatenate([hrT_t, outT], axis=0)
    yT = jnp.dot(Wd2[:, :],
                 _lrelu(jnp.dot(Wd1[:, :], dec_inT) + bd1c[:, :])) + bd2c[:, :]
    o_ref[0, :, :] = yT


def _pad_rows(a, rows):
    return jnp.zeros((rows, a.shape[1]), a.dtype).at[:a.shape[0]].set(a)


def _pad_cols(a, cols):
    return jnp.zeros((a.shape[0], cols), a.dtype).at[:, :a.shape[1]].set(a)


def kernel(robot_embedding, obstacle_embedding, params):
    p = params
    f32 = jnp.float32

    x_all = jnp.concatenate([robot_embedding, obstacle_embedding], axis=0)
    x_all = jnp.concatenate(
        [x_all, jnp.zeros((NS, 3), f32)], axis=1)
    x_allT = x_all.T

    We1 = _pad_rows(p['emb1_W'].T, 8)
    be1 = p['emb1_b'].reshape(1, -1)
    We2 = p['emb2_W'].T
    be2 = p['emb2_b'].reshape(1, -1)
    We1T = _pad_cols(p['emb1_W'], 8)
    be1c = p['emb1_b'].reshape(-1, 1)
    We2T = p['emb2_W']
    be2c = p['emb2_b'].reshape(-1, 1)
    Wo1T = _pad_cols(p['oemb1_W'], 8)
    bo1c = p['oemb1_b'].reshape(-1, 1)
    Wo2T = p['oemb2_W']
    bo2c = p['oemb2_b'].reshape(-1, 1)

    Wq = p['q_W'].T
    Wa1q = p['attn1_W'][:, :EMB].T
    ba1 = p['attn1_b'].reshape(1, -1)

    Wk_r = _pad_rows(p['k_W'][:, 5:].T, 8)
    Wv_r = _pad_rows(p['v_W'][:, 5:].T, 8)
    bv = p['v_b'].reshape(1, -1)
    Whr_rec = _pad_rows(p['hard1_W'][:, EMB:EMB + 5].T, 8)
    Who_rec = _pad_rows(p['hardo1_W'][:, EMB:EMB + 5].T, 8)

    Wk_sT = _pad_cols(p['k_W'][:, :5], 8)
    Wv_sT = _pad_cols(p['v_W'][:, :5], 8)
    Whard1h = p['hard1_W'][:, :EMB]
    Whr_xT = jnp.zeros((EMB, 8), f32).at[:, 3:5].set(p['hard1_W'][:, EMB + 5:])
    bhrc = p['hard1_b'].reshape(-1, 1)
    Whardo1h = p['hardo1_W'][:, :EMB]
    bhoc = p['hardo1_b'].reshape(-1, 1)

    A1k = p['attn1_W'][:, EMB:]
    A1k_bd = jnp.zeros((2 * EMB, 2 * EMB), f32)
    A1k_bd = A1k_bd.at[:EMB, :EMB].set(A1k).at[EMB:, EMB:].set(A1k)
    a2row = p['attn2_W']
    a2_2 = jnp.zeros((2, 2 * EMB), f32)
    a2_2 = a2_2.at[0:1, :EMB].set(a2row).at[1:2, EMB:].set(a2row)

    d_r = p['henc_W'][1] - p['henc_W'][0]
    wrr = (p['hard2_W'].T @ d_r).reshape(1, EMB)
    crr = (p['hard2_b'] @ d_r + p['henc_b'][1] - p['henc_b'][0]).reshape(1, 1)
    d_o = p['henco_W'][1] - p['henco_W'][0]
    wor = (p['hardo2_W'].T @ d_o).reshape(1, EMB)
    cor = (p['hardo2_b'] @ d_o + p['henco_b'][1] - p['henco_b'][0]).reshape(1, 1)
    wrr2 = jnp.zeros((2, 2 * EMB), f32)
    wrr2 = wrr2.at[0:1, :EMB].set(wrr).at[1:2, EMB:].set(wrr)
    wor2 = jnp.zeros((2, 2 * EMB), f32)
    wor2 = wor2.at[0:1, :EMB].set(wor).at[1:2, EMB:].set(wor)

    Wd1 = p['dec1_W']
    bd1c = p['dec1_b'].reshape(-1, 1)
    Wd2 = p['dec2_W']
    bd2c = p['dec2_b'].reshape(-1, 1)

    inputs = [x_all, x_allT,
              We1, be1, We2, be2,
              We1T, be1c, We2T, be2c,
              Wo1T, bo1c, Wo2T, bo2c,
              Wq, Wa1q, ba1,
              Wk_r, Wv_r, bv, Whr_rec, Who_rec,
              Wk_sT, Wv_sT,
              Whard1h, Whr_xT, bhrc,
              Whardo1h, bhoc,
              A1k_bd, a2_2, wrr2, crr, wor2, cor,
              Wd1, bd1c, Wd2, bd2c]

    def rep_spec(a):
        nd = a.ndim
        return pl.BlockSpec(a.shape, lambda i, _nd=nd: (0,) * _nd)

    yT = pl.pallas_call(
        _fused,
        grid=(GRID,),
        in_specs=[rep_spec(a) for a in inputs],
        out_specs=pl.BlockSpec((1, 2 * EMB, TR), lambda i: (i, 0, 0)),
        out_shape=jax.ShapeDtypeStruct((GRID, 2 * EMB, TR), f32),
        scratch_shapes=[
            pltpu.VMEM((NR, EMB), f32),
            pltpu.VMEM((NR, EMB), f32),
            pltpu.VMEM((NR, EMB), f32),
            pltpu.VMEM((NR, EMB), f32),
            pltpu.VMEM((NR, EMB), f32),
            pltpu.VMEM((NR, EMB), f32),
            pltpu.VMEM((2 * EMB, NS), f32),
            pltpu.VMEM((2 * EMB, NS), f32),
            pltpu.VMEM((2 * EMB, NR), f32),
            pltpu.VMEM((2 * EMB, NO), f32),
        ],
        compiler_params=pltpu.CompilerParams(
            dimension_semantics=("arbitrary",)),
    )(*inputs)
    return yT.transpose(0, 2, 1).reshape(NR, 2 * EMB)


# speedup vs baseline: 1.1428x; 1.0657x over previous
import jax
import jax.numpy as jnp
from jax.experimental import pallas as pl
from jax.experimental.pallas import tpu as pltpu

EMB = 64
NR = 256
NO = 1024
NS = NR + NO
TR = 32
GRID = NR // TR

def _lrelu(x):
    return jnp.maximum(x, 0.01 * x)


_relu = jax.nn.relu


def _fused(x_all_ref, x_allT_ref,
           We1, be1, We2, be2,
           We1T, be1c, We2T, be2c,
           Wo1T, bo1c, Wo2T, bo2c,
           Wq, Wa1q, ba1,
           Wk_r, Wv_r, bv, Whr_rec, Who_rec,
           Wk_sT, Wv_sT,
           Whard1h, Whr_xT, bhrc,
           Whardo1h, bhoc,
           A1k_bd, a2_2, wrr2, crr, wor2, cor,
           Wd1, bd1c, Wd2, bd2c,
           o_ref,
           hr_s, A1q_s, Ki_s, Vi_s, Rgr_s, Rgo_s,
           KjT2_s, VjT2_s,
           SgrT2_s, SgoT2_s):
    pid = pl.program_id(0)

    @pl.when(pid == 0)
    def _prep():
        x_all = x_all_ref[:, :]
        x_r = x_all[:NR]
        x_allT = x_allT_ref[:, :]
        x_rT = x_allT[:, :NR]
        x_oT = x_allT[:, NR:]

        h_r = _lrelu(jnp.dot(_lrelu(jnp.dot(x_r, We1[:, :]) + be1[:, :]),
                             We2[:, :]) + be2[:, :])
        hr_s[:, :] = h_r
        q = jnp.dot(h_r, Wq[:, :])
        A1q_s[:, :] = jnp.dot(q, Wa1q[:, :]) + ba1[:, :]
        Ki_s[:, :] = jnp.dot(x_r, Wk_r[:, :])
        Vi_s[:, :] = jnp.dot(x_r, Wv_r[:, :]) + bv[:, :]
        Rgr_s[:, :] = jnp.dot(x_r, Whr_rec[:, :])
        Rgo_s[:, :] = jnp.dot(x_r, Who_rec[:, :])

        h_rT = _lrelu(jnp.dot(We2T[:, :],
                              _lrelu(jnp.dot(We1T[:, :], x_rT) + be1c[:, :]))
                      + be2c[:, :])
        h_oT = _lrelu(jnp.dot(Wo2T[:, :],
                              _lrelu(jnp.dot(Wo1T[:, :], x_oT) + bo1c[:, :]))
                      + bo2c[:, :])
        kjT = jnp.dot(Wk_sT[:, :], x_allT)
        KjT2_s[:, :] = jnp.concatenate([kjT, kjT], axis=0)
        vjT = jnp.dot(Wv_sT[:, :], x_allT)
        VjT2_s[:, :] = jnp.concatenate([vjT, vjT], axis=0)
        sgrT = (jnp.dot(Whard1h[:, :], h_rT)
                + jnp.dot(Whr_xT[:, :], x_rT) + bhrc[:, :])
        SgrT2_s[:, :] = jnp.concatenate([sgrT, sgrT], axis=0)
        sgoT = jnp.dot(Whardo1h[:, :], h_oT) + bhoc[:, :]
        SgoT2_s[:, :] = jnp.concatenate([sgoT, sgoT], axis=0)

    i0 = pid * TR
    hrT_t = jnp.transpose(hr_s[pl.ds(i0, TR), :])
    A1qT_t = jnp.transpose(A1q_s[pl.ds(i0, TR), :])
    KiT_t = jnp.transpose(Ki_s[pl.ds(i0, TR), :])
    ViT_t = jnp.transpose(Vi_s[pl.ds(i0, TR), :])
    RgrT_t = jnp.transpose(Rgr_s[pl.ds(i0, TR), :])
    RgoT_t = jnp.transpose(Rgo_s[pl.ds(i0, TR), :])

    KjT2 = KjT2_s[:, :]
    VjT2 = VjT2_s[:, :]
    SgrT2 = SgrT2_s[:, :]
    SgoT2 = SgoT2_s[:, :]

    def cols2(a, s):
        return jnp.concatenate(
            [a[:, 2 * s:2 * s + 1], a[:, 2 * s + 1:2 * s + 2]], axis=0)

    outs = []
    for s in range(TR // 2):
        ke2 = _lrelu(cols2(KiT_t, s) + KjT2)
        pre2 = _relu(jnp.dot(A1k_bd[:, :], ke2,
                             preferred_element_type=jnp.float32)
                     + cols2(A1qT_t, s))
        sc2 = jnp.dot(a2_2[:, :], pre2,
                      preferred_element_type=jnp.float32)

        pgr2 = _relu(SgrT2 + cols2(RgrT_t, s))
        gr2 = jnp.dot(wrr2[:, :], pgr2,
                      preferred_element_type=jnp.float32) + crr[:, :]
        pgo2 = _relu(SgoT2 + cols2(RgoT_t, s))
        go2 = jnp.dot(wor2[:, :], pgo2,
                      preferred_element_type=jnp.float32) + cor[:, :]
        gate2 = jax.nn.sigmoid(jnp.concatenate([gr2, go2], axis=1))

        m2 = jnp.max(sc2, axis=1, keepdims=True)
        ex2 = jnp.exp(sc2 - m2)
        den2 = jnp.sum(ex2, axis=1, keepdims=True)
        u2 = ex2 * gate2

        ve2 = _lrelu(cols2(ViT_t, s) + VjT2)
        o_a = jnp.sum(ve2[:EMB] * u2[0:1], axis=1, keepdims=True)
        o_b = jnp.sum(ve2[EMB:] * u2[1:2], axis=1, keepdims=True)
        outs.append(o_a / (den2[0:1] + 1e-16))
        outs.append(o_b / (den2[1:2] + 1e-16))

    outT = jnp.concatenate(outs, axis=1)
    dec_inT = jnp.concatenate([hrT_t, outT], axis=0)
    yT = jnp.dot(Wd2[:, :],
                 _lrelu(jnp.dot(Wd1[:, :], dec_inT) + bd1c[:, :])) + bd2c[:, :]
    o_ref[0, :, :] = yT


def _pad_rows(a, rows):
    return jnp.zeros((rows, a.shape[1]), a.dtype).at[:a.shape[0]].set(a)


def _pad_cols(a, cols):
    return jnp.zeros((a.shape[0], cols), a.dtype).at[:, :a.shape[1]].set(a)


def kernel(robot_embedding, obstacle_embedding, params):
    p = params
    f32 = jnp.float32

    x_all = jnp.concatenate([robot_embedding, obstacle_embedding], axis=0)
    x_all = jnp.concatenate(
        [x_all, jnp.zeros((NS, 3), f32)], axis=1)
    x_allT = x_all.T

    We1 = _pad_rows(p['emb1_W'].T, 8)
    be1 = p['emb1_b'].reshape(1, -1)
    We2 = p['emb2_W'].T
    be2 = p['emb2_b'].reshape(1, -1)
    We1T = _pad_cols(p['emb1_W'], 8)
    be1c = p['emb1_b'].reshape(-1, 1)
    We2T = p['emb2_W']
    be2c = p['emb2_b'].reshape(-1, 1)
    Wo1T = _pad_cols(p['oemb1_W'], 8)
    bo1c = p['oemb1_b'].reshape(-1, 1)
    Wo2T = p['oemb2_W']
    bo2c = p['oemb2_b'].reshape(-1, 1)

    Wq = p['q_W'].T
    Wa1q = p['attn1_W'][:, :EMB].T
    ba1 = p['attn1_b'].reshape(1, -1)

    Wk_r = _pad_rows(p['k_W'][:, 5:].T, 8)
    Wv_r = _pad_rows(p['v_W'][:, 5:].T, 8)
    bv = p['v_b'].reshape(1, -1)
    Whr_rec = _pad_rows(p['hard1_W'][:, EMB:EMB + 5].T, 8)
    Who_rec = _pad_rows(p['hardo1_W'][:, EMB:EMB + 5].T, 8)

    Wk_sT = _pad_cols(p['k_W'][:, :5], 8)
    Wv_sT = _pad_cols(p['v_W'][:, :5], 8)
    Whard1h = p['hard1_W'][:, :EMB]
    Whr_xT = jnp.zeros((EMB, 8), f32).at[:, 3:5].set(p['hard1_W'][:, EMB + 5:])
    bhrc = p['hard1_b'].reshape(-1, 1)
    Whardo1h = p['hardo1_W'][:, :EMB]
    bhoc = p['hardo1_b'].reshape(-1, 1)

    A1k = p['attn1_W'][:, EMB:]
    A1k_bd = jnp.zeros((2 * EMB, 2 * EMB), f32)
    A1k_bd = A1k_bd.at[:EMB, :EMB].set(A1k).at[EMB:, EMB:].set(A1k)
    a2row = p['attn2_W']
    a2_2 = jnp.zeros((2, 2 * EMB), f32)
    a2_2 = a2_2.at[0:1, :EMB].set(a2row).at[1:2, EMB:].set(a2row)

    d_r = p['henc_W'][1] - p['henc_W'][0]
    wrr = (p['hard2_W'].T @ d_r).reshape(1, EMB)
    crr = (p['hard2_b'] @ d_r + p['henc_b'][1] - p['henc_b'][0]).reshape(1, 1)
    d_o = p['henco_W'][1] - p['henco_W'][0]
    wor = (p['hardo2_W'].T @ d_o).reshape(1, EMB)
    cor = (p['hardo2_b'] @ d_o + p['henco_b'][1] - p['henco_b'][0]).reshape(1, 1)
    wrr2 = jnp.zeros((2, 2 * EMB), f32)
    wrr2 = wrr2.at[0:1, :EMB].set(wrr).at[1:2, EMB:].set(wrr)
    wor2 = jnp.zeros((2, 2 * EMB), f32)
    wor2 = wor2.at[0:1, :EMB].set(wor).at[1:2, EMB:].set(wor)

    Wd1 = p['dec1_W']
    bd1c = p['dec1_b'].reshape(-1, 1)
    Wd2 = p['dec2_W']
    bd2c = p['dec2_b'].reshape(-1, 1)

    inputs = [x_all, x_allT,
              We1, be1, We2, be2,
              We1T, be1c, We2T, be2c,
              Wo1T, bo1c, Wo2T, bo2c,
              Wq, Wa1q, ba1,
              Wk_r, Wv_r, bv, Whr_rec, Who_rec,
              Wk_sT, Wv_sT,
              Whard1h, Whr_xT, bhrc,
              Whardo1h, bhoc,
              A1k_bd, a2_2, wrr2, crr, wor2, cor,
              Wd1, bd1c, Wd2, bd2c]

    def rep_spec(a):
        nd = a.ndim
        return pl.BlockSpec(a.shape, lambda i, _nd=nd: (0,) * _nd)

    yT = pl.pallas_call(
        _fused,
        grid=(GRID,),
        in_specs=[rep_spec(a) for a in inputs],
        out_specs=pl.BlockSpec((1, 2 * EMB, TR), lambda i: (i, 0, 0)),
        out_shape=jax.ShapeDtypeStruct((GRID, 2 * EMB, TR), f32),
        scratch_shapes=[
            pltpu.VMEM((NR, EMB), f32),
            pltpu.VMEM((NR, EMB), f32),
            pltpu.VMEM((NR, EMB), f32),
            pltpu.VMEM((NR, EMB), f32),
            pltpu.VMEM((NR, EMB), f32),
            pltpu.VMEM((NR, EMB), f32),
            pltpu.VMEM((2 * EMB, NS), f32),
            pltpu.VMEM((2 * EMB, NS), f32),
            pltpu.VMEM((2 * EMB, NR), f32),
            pltpu.VMEM((2 * EMB, NO), f32),
        ],
        compiler_params=pltpu.CompilerParams(
            dimension_semantics=("arbitrary",)),
    )(*inputs)
    return yT.transpose(0, 2, 1).reshape(NR, 2 * EMB)


# speedup vs baseline: 1.1971x; 1.0475x over previous
import jax
import jax.numpy as jnp
from jax.experimental import pallas as pl
from jax.experimental.pallas import tpu as pltpu

EMB = 64
NR = 256
NO = 1024
NS = NR + NO
TR = 64
GRID = NR // TR

def _lrelu(x):
    return jnp.maximum(x, 0.01 * x)


_relu = jax.nn.relu


def _fused(x_all_ref, x_allT_ref,
           We1, be1, We2, be2,
           We1T, be1c, We2T, be2c,
           Wo1T, bo1c, Wo2T, bo2c,
           Wq, Wa1q, ba1,
           Wk_r, Wv_r, bv, Whr_rec, Who_rec,
           Wk_sT, Wv_sT,
           Whard1h, Whr_xT, bhrc,
           Whardo1h, bhoc,
           A1k_bd, a2_2, wrr2, crr, wor2, cor,
           Wd1, bd1c, Wd2, bd2c,
           o_ref,
           hr_s, A1q_s, Ki_s, Vi_s, Rgr_s, Rgo_s,
           KjT2_s, VjT2_s,
           SgrT2_s, SgoT2_s):
    pid = pl.program_id(0)

    @pl.when(pid == 0)
    def _prep():
        x_all = x_all_ref[:, :]
        x_r = x_all[:NR]
        x_allT = x_allT_ref[:, :]
        x_rT = x_allT[:, :NR]
        x_oT = x_allT[:, NR:]

        h_r = _lrelu(jnp.dot(_lrelu(jnp.dot(x_r, We1[:, :]) + be1[:, :]),
                             We2[:, :]) + be2[:, :])
        hr_s[:, :] = h_r
        q = jnp.dot(h_r, Wq[:, :])
        A1q_s[:, :] = jnp.dot(q, Wa1q[:, :]) + ba1[:, :]
        Ki_s[:, :] = jnp.dot(x_r, Wk_r[:, :])
        Vi_s[:, :] = jnp.dot(x_r, Wv_r[:, :]) + bv[:, :]
        Rgr_s[:, :] = jnp.dot(x_r, Whr_rec[:, :])
        Rgo_s[:, :] = jnp.dot(x_r, Who_rec[:, :])

        h_rT = _lrelu(jnp.dot(We2T[:, :],
                              _lrelu(jnp.dot(We1T[:, :], x_rT) + be1c[:, :]))
                      + be2c[:, :])
        h_oT = _lrelu(jnp.dot(Wo2T[:, :],
                              _lrelu(jnp.dot(Wo1T[:, :], x_oT) + bo1c[:, :]))
                      + bo2c[:, :])
        kjT = jnp.dot(Wk_sT[:, :], x_allT)
        KjT2_s[:, :] = jnp.concatenate([kjT, kjT], axis=0)
        vjT = jnp.dot(Wv_sT[:, :], x_allT)
        VjT2_s[:, :] = jnp.concatenate([vjT, vjT], axis=0)
        sgrT = (jnp.dot(Whard1h[:, :], h_rT)
                + jnp.dot(Whr_xT[:, :], x_rT) + bhrc[:, :])
        SgrT2_s[:, :] = jnp.concatenate([sgrT, sgrT], axis=0)
        sgoT = jnp.dot(Whardo1h[:, :], h_oT) + bhoc[:, :]
        SgoT2_s[:, :] = jnp.concatenate([sgoT, sgoT], axis=0)

    i0 = pid * TR
    hrT_t = jnp.transpose(hr_s[pl.ds(i0, TR), :])
    A1qT_t = jnp.transpose(A1q_s[pl.ds(i0, TR), :])
    KiT_t = jnp.transpose(Ki_s[pl.ds(i0, TR), :])
    ViT_t = jnp.transpose(Vi_s[pl.ds(i0, TR), :])
    RgrT_t = jnp.transpose(Rgr_s[pl.ds(i0, TR), :])
    RgoT_t = jnp.transpose(Rgo_s[pl.ds(i0, TR), :])

    KjT2 = KjT2_s[:, :]
    VjT2 = VjT2_s[:, :]
    SgrT2 = SgrT2_s[:, :]
    SgoT2 = SgoT2_s[:, :]

    def cols2(a, s):
        return jnp.concatenate(
            [a[:, 2 * s:2 * s + 1], a[:, 2 * s + 1:2 * s + 2]], axis=0)

    outs = []
    for s in range(TR // 2):
        ke2 = _lrelu(cols2(KiT_t, s) + KjT2)
        pre2 = _relu(jnp.dot(A1k_bd[:, :], ke2,
                             preferred_element_type=jnp.float32)
                     + cols2(A1qT_t, s))
        sc2 = jnp.dot(a2_2[:, :], pre2,
                      preferred_element_type=jnp.float32)

        pgr2 = _relu(SgrT2 + cols2(RgrT_t, s))
        gr2 = jnp.dot(wrr2[:, :], pgr2,
                      preferred_element_type=jnp.float32) + crr[:, :]
        pgo2 = _relu(SgoT2 + cols2(RgoT_t, s))
        go2 = jnp.dot(wor2[:, :], pgo2,
                      preferred_element_type=jnp.float32) + cor[:, :]
        gate2 = jax.nn.sigmoid(jnp.concatenate([gr2, go2], axis=1))

        m2 = jnp.max(sc2, axis=1, keepdims=True)
        ex2 = jnp.exp(sc2 - m2)
        den2 = jnp.sum(ex2, axis=1, keepdims=True)
        u2 = ex2 * gate2

        ve2 = _lrelu(cols2(ViT_t, s) + VjT2)
        o_a = jnp.sum(ve2[:EMB] * u2[0:1], axis=1, keepdims=True)
        o_b = jnp.sum(ve2[EMB:] * u2[1:2], axis=1, keepdims=True)
        outs.append(o_a / (den2[0:1] + 1e-16))
        outs.append(o_b / (den2[1:2] + 1e-16))

    outT = jnp.concatenate(outs, axis=1)
    dec_inT = jnp.concatenate([hrT_t, outT], axis=0)
    yT = jnp.dot(Wd2[:, :],
                 _lrelu(jnp.dot(Wd1[:, :], dec_inT) + bd1c[:, :])) + bd2c[:, :]
    o_ref[0, :, :] = yT


def _pad_rows(a, rows):
    return jnp.zeros((rows, a.shape[1]), a.dtype).at[:a.shape[0]].set(a)


def _pad_cols(a, cols):
    return jnp.zeros((a.shape[0], cols), a.dtype).at[:, :a.shape[1]].set(a)


def kernel(robot_embedding, obstacle_embedding, params):
    p = params
    f32 = jnp.float32

    x_all = jnp.concatenate([robot_embedding, obstacle_embedding], axis=0)
    x_all = jnp.concatenate(
        [x_all, jnp.zeros((NS, 3), f32)], axis=1)
    x_allT = x_all.T

    We1 = _pad_rows(p['emb1_W'].T, 8)
    be1 = p['emb1_b'].reshape(1, -1)
    We2 = p['emb2_W'].T
    be2 = p['emb2_b'].reshape(1, -1)
    We1T = _pad_cols(p['emb1_W'], 8)
    be1c = p['emb1_b'].reshape(-1, 1)
    We2T = p['emb2_W']
    be2c = p['emb2_b'].reshape(-1, 1)
    Wo1T = _pad_cols(p['oemb1_W'], 8)
    bo1c = p['oemb1_b'].reshape(-1, 1)
    Wo2T = p['oemb2_W']
    bo2c = p['oemb2_b'].reshape(-1, 1)

    Wq = p['q_W'].T
    Wa1q = p['attn1_W'][:, :EMB].T
    ba1 = p['attn1_b'].reshape(1, -1)

    Wk_r = _pad_rows(p['k_W'][:, 5:].T, 8)
    Wv_r = _pad_rows(p['v_W'][:, 5:].T, 8)
    bv = p['v_b'].reshape(1, -1)
    Whr_rec = _pad_rows(p['hard1_W'][:, EMB:EMB + 5].T, 8)
    Who_rec = _pad_rows(p['hardo1_W'][:, EMB:EMB + 5].T, 8)

    Wk_sT = _pad_cols(p['k_W'][:, :5], 8)
    Wv_sT = _pad_cols(p['v_W'][:, :5], 8)
    Whard1h = p['hard1_W'][:, :EMB]
    Whr_xT = jnp.zeros((EMB, 8), f32).at[:, 3:5].set(p['hard1_W'][:, EMB + 5:])
    bhrc = p['hard1_b'].reshape(-1, 1)
    Whardo1h = p['hardo1_W'][:, :EMB]
    bhoc = p['hardo1_b'].reshape(-1, 1)

    A1k = p['attn1_W'][:, EMB:]
    A1k_bd = jnp.zeros((2 * EMB, 2 * EMB), f32)
    A1k_bd = A1k_bd.at[:EMB, :EMB].set(A1k).at[EMB:, EMB:].set(A1k)
    a2row = p['attn2_W']
    a2_2 = jnp.zeros((2, 2 * EMB), f32)
    a2_2 = a2_2.at[0:1, :EMB].set(a2row).at[1:2, EMB:].set(a2row)

    d_r = p['henc_W'][1] - p['henc_W'][0]
    wrr = (p['hard2_W'].T @ d_r).reshape(1, EMB)
    crr = (p['hard2_b'] @ d_r + p['henc_b'][1] - p['henc_b'][0]).reshape(1, 1)
    d_o = p['henco_W'][1] - p['henco_W'][0]
    wor = (p['hardo2_W'].T @ d_o).reshape(1, EMB)
    cor = (p['hardo2_b'] @ d_o + p['henco_b'][1] - p['henco_b'][0]).reshape(1, 1)
    wrr2 = jnp.zeros((2, 2 * EMB), f32)
    wrr2 = wrr2.at[0:1, :EMB].set(wrr).at[1:2, EMB:].set(wrr)
    wor2 = jnp.zeros((2, 2 * EMB), f32)
    wor2 = wor2.at[0:1, :EMB].set(wor).at[1:2, EMB:].set(wor)

    Wd1 = p['dec1_W']
    bd1c = p['dec1_b'].reshape(-1, 1)
    Wd2 = p['dec2_W']
    bd2c = p['dec2_b'].reshape(-1, 1)

    inputs = [x_all, x_allT,
              We1, be1, We2, be2,
              We1T, be1c, We2T, be2c,
              Wo1T, bo1c, Wo2T, bo2c,
              Wq, Wa1q, ba1,
              Wk_r, Wv_r, bv, Whr_rec, Who_rec,
              Wk_sT, Wv_sT,
              Whard1h, Whr_xT, bhrc,
              Whardo1h, bhoc,
              A1k_bd, a2_2, wrr2, crr, wor2, cor,
              Wd1, bd1c, Wd2, bd2c]

    def rep_spec(a):
        nd = a.ndim
        return pl.BlockSpec(a.shape, lambda i, _nd=nd: (0,) * _nd)

    yT = pl.pallas_call(
        _fused,
        grid=(GRID,),
        in_specs=[rep_spec(a) for a in inputs],
        out_specs=pl.BlockSpec((1, 2 * EMB, TR), lambda i: (i, 0, 0)),
        out_shape=jax.ShapeDtypeStruct((GRID, 2 * EMB, TR), f32),
        scratch_shapes=[
            pltpu.VMEM((NR, EMB), f32),
            pltpu.VMEM((NR, EMB), f32),
            pltpu.VMEM((NR, EMB), f32),
            pltpu.VMEM((NR, EMB), f32),
            pltpu.VMEM((NR, EMB), f32),
            pltpu.VMEM((NR, EMB), f32),
            pltpu.VMEM((2 * EMB, NS), f32),
            pltpu.VMEM((2 * EMB, NS), f32),
            pltpu.VMEM((2 * EMB, NR), f32),
            pltpu.VMEM((2 * EMB, NO), f32),
        ],
        compiler_params=pltpu.CompilerParams(
            dimension_semantics=("arbitrary",)),
    )(*inputs)
    return yT.transpose(0, 2, 1).reshape(NR, 2 * EMB)


# speedup vs baseline: 1.2033x; 1.0052x over previous
import jax
import jax.numpy as jnp
from jax.experimental import pallas as pl
from jax.experimental.pallas import tpu as pltpu

EMB = 64
NR = 256
NO = 1024
NS = NR + NO
TR = 128
GRID = NR // TR

def _lrelu(x):
    return jnp.maximum(x, 0.01 * x)


_relu = jax.nn.relu


def _fused(x_all_ref, x_allT_ref,
           We1, be1, We2, be2,
           We1T, be1c, We2T, be2c,
           Wo1T, bo1c, Wo2T, bo2c,
           Wq, Wa1q, ba1,
           Wk_r, Wv_r, bv, Whr_rec, Who_rec,
           Wk_sT, Wv_sT,
           Whard1h, Whr_xT, bhrc,
           Whardo1h, bhoc,
           A1k_bd, a2_2, wrr2, crr, wor2, cor,
           Wd1, bd1c, Wd2, bd2c,
           o_ref,
           hr_s, A1q_s, Ki_s, Vi_s, Rgr_s, Rgo_s,
           KjT2_s, VjT2_s,
           SgrT2_s, SgoT2_s):
    pid = pl.program_id(0)

    @pl.when(pid == 0)
    def _prep():
        x_all = x_all_ref[:, :]
        x_r = x_all[:NR]
        x_allT = x_allT_ref[:, :]
        x_rT = x_allT[:, :NR]
        x_oT = x_allT[:, NR:]

        h_r = _lrelu(jnp.dot(_lrelu(jnp.dot(x_r, We1[:, :]) + be1[:, :]),
                             We2[:, :]) + be2[:, :])
        hr_s[:, :] = h_r
        q = jnp.dot(h_r, Wq[:, :])
        A1q_s[:, :] = jnp.dot(q, Wa1q[:, :]) + ba1[:, :]
        Ki_s[:, :] = jnp.dot(x_r, Wk_r[:, :])
        Vi_s[:, :] = jnp.dot(x_r, Wv_r[:, :]) + bv[:, :]
        Rgr_s[:, :] = jnp.dot(x_r, Whr_rec[:, :])
        Rgo_s[:, :] = jnp.dot(x_r, Who_rec[:, :])

        h_rT = _lrelu(jnp.dot(We2T[:, :],
                              _lrelu(jnp.dot(We1T[:, :], x_rT) + be1c[:, :]))
                      + be2c[:, :])
        h_oT = _lrelu(jnp.dot(Wo2T[:, :],
                              _lrelu(jnp.dot(Wo1T[:, :], x_oT) + bo1c[:, :]))
                      + bo2c[:, :])
        kjT = jnp.dot(Wk_sT[:, :], x_allT)
        KjT2_s[:, :] = jnp.concatenate([kjT, kjT], axis=0)
        vjT = jnp.dot(Wv_sT[:, :], x_allT)
        VjT2_s[:, :] = jnp.concatenate([vjT, vjT], axis=0)
        sgrT = (jnp.dot(Whard1h[:, :], h_rT)
                + jnp.dot(Whr_xT[:, :], x_rT) + bhrc[:, :])
        SgrT2_s[:, :] = jnp.concatenate([sgrT, sgrT], axis=0)
        sgoT = jnp.dot(Whardo1h[:, :], h_oT) + bhoc[:, :]
        SgoT2_s[:, :] = jnp.concatenate([sgoT, sgoT], axis=0)

    i0 = pid * TR
    hrT_t = jnp.transpose(hr_s[pl.ds(i0, TR), :])
    A1qT_t = jnp.transpose(A1q_s[pl.ds(i0, TR), :])
    KiT_t = jnp.transpose(Ki_s[pl.ds(i0, TR), :])
    ViT_t = jnp.transpose(Vi_s[pl.ds(i0, TR), :])
    RgrT_t = jnp.transpose(Rgr_s[pl.ds(i0, TR), :])
    RgoT_t = jnp.transpose(Rgo_s[pl.ds(i0, TR), :])

    KjT2 = KjT2_s[:, :]
    VjT2 = VjT2_s[:, :]
    SgrT2 = SgrT2_s[:, :]
    SgoT2 = SgoT2_s[:, :]

    def cols2(a, s):
        return jnp.concatenate(
            [a[:, 2 * s:2 * s + 1], a[:, 2 * s + 1:2 * s + 2]], axis=0)

    outs = []
    for s in range(TR // 2):
        ke2 = _lrelu(cols2(KiT_t, s) + KjT2)
        pre2 = _relu(jnp.dot(A1k_bd[:, :], ke2,
                             preferred_element_type=jnp.float32)
                     + cols2(A1qT_t, s))
        sc2 = jnp.dot(a2_2[:, :], pre2,
                      preferred_element_type=jnp.float32)

        pgr2 = _relu(SgrT2 + cols2(RgrT_t, s))
        gr2 = jnp.dot(wrr2[:, :], pgr2,
                      preferred_element_type=jnp.float32) + crr[:, :]
        pgo2 = _relu(SgoT2 + cols2(RgoT_t, s))
        go2 = jnp.dot(wor2[:, :], pgo2,
                      preferred_element_type=jnp.float32) + cor[:, :]
        gate2 = jax.nn.sigmoid(jnp.concatenate([gr2, go2], axis=1))

        m2 = jnp.max(sc2, axis=1, keepdims=True)
        ex2 = jnp.exp(sc2 - m2)
        den2 = jnp.sum(ex2, axis=1, keepdims=True)
        u2 = ex2 * gate2

        ve2 = _lrelu(cols2(ViT_t, s) + VjT2)
        o_a = jnp.sum(ve2[:EMB] * u2[0:1], axis=1, keepdims=True)
        o_b = jnp.sum(ve2[EMB:] * u2[1:2], axis=1, keepdims=True)
        outs.append(o_a / (den2[0:1] + 1e-16))
        outs.append(o_b / (den2[1:2] + 1e-16))

    outT = jnp.concatenate(outs, axis=1)
    dec_inT = jnp.concatenate([hrT_t, outT], axis=0)
    yT = jnp.dot(Wd2[:, :],
                 _lrelu(jnp.dot(Wd1[:, :], dec_inT) + bd1c[:, :])) + bd2c[:, :]
    o_ref[0, :, :] = yT


def _pad_rows(a, rows):
    return jnp.zeros((rows, a.shape[1]), a.dtype).at[:a.shape[0]].set(a)


def _pad_cols(a, cols):
    return jnp.zeros((a.shape[0], cols), a.dtype).at[:, :a.shape[1]].set(a)


def kernel(robot_embedding, obstacle_embedding, params):
    p = params
    f32 = jnp.float32

    x_all = jnp.concatenate([robot_embedding, obstacle_embedding], axis=0)
    x_all = jnp.concatenate(
        [x_all, jnp.zeros((NS, 3), f32)], axis=1)
    x_allT = x_all.T

    We1 = _pad_rows(p['emb1_W'].T, 8)
    be1 = p['emb1_b'].reshape(1, -1)
    We2 = p['emb2_W'].T
    be2 = p['emb2_b'].reshape(1, -1)
    We1T = _pad_cols(p['emb1_W'], 8)
    be1c = p['emb1_b'].reshape(-1, 1)
    We2T = p['emb2_W']
    be2c = p['emb2_b'].reshape(-1, 1)
    Wo1T = _pad_cols(p['oemb1_W'], 8)
    bo1c = p['oemb1_b'].reshape(-1, 1)
    Wo2T = p['oemb2_W']
    bo2c = p['oemb2_b'].reshape(-1, 1)

    Wq = p['q_W'].T
    Wa1q = p['attn1_W'][:, :EMB].T
    ba1 = p['attn1_b'].reshape(1, -1)

    Wk_r = _pad_rows(p['k_W'][:, 5:].T, 8)
    Wv_r = _pad_rows(p['v_W'][:, 5:].T, 8)
    bv = p['v_b'].reshape(1, -1)
    Whr_rec = _pad_rows(p['hard1_W'][:, EMB:EMB + 5].T, 8)
    Who_rec = _pad_rows(p['hardo1_W'][:, EMB:EMB + 5].T, 8)

    Wk_sT = _pad_cols(p['k_W'][:, :5], 8)
    Wv_sT = _pad_cols(p['v_W'][:, :5], 8)
    Whard1h = p['hard1_W'][:, :EMB]
    Whr_xT = jnp.zeros((EMB, 8), f32).at[:, 3:5].set(p['hard1_W'][:, EMB + 5:])
    bhrc = p['hard1_b'].reshape(-1, 1)
    Whardo1h = p['hardo1_W'][:, :EMB]
    bhoc = p['hardo1_b'].reshape(-1, 1)

    A1k = p['attn1_W'][:, EMB:]
    A1k_bd = jnp.zeros((2 * EMB, 2 * EMB), f32)
    A1k_bd = A1k_bd.at[:EMB, :EMB].set(A1k).at[EMB:, EMB:].set(A1k)
    a2row = p['attn2_W']
    a2_2 = jnp.zeros((2, 2 * EMB), f32)
    a2_2 = a2_2.at[0:1, :EMB].set(a2row).at[1:2, EMB:].set(a2row)

    d_r = p['henc_W'][1] - p['henc_W'][0]
    wrr = (p['hard2_W'].T @ d_r).reshape(1, EMB)
    crr = (p['hard2_b'] @ d_r + p['henc_b'][1] - p['henc_b'][0]).reshape(1, 1)
    d_o = p['henco_W'][1] - p['henco_W'][0]
    wor = (p['hardo2_W'].T @ d_o).reshape(1, EMB)
    cor = (p['hardo2_b'] @ d_o + p['henco_b'][1] - p['henco_b'][0]).reshape(1, 1)
    wrr2 = jnp.zeros((2, 2 * EMB), f32)
    wrr2 = wrr2.at[0:1, :EMB].set(wrr).at[1:2, EMB:].set(wrr)
    wor2 = jnp.zeros((2, 2 * EMB), f32)
    wor2 = wor2.at[0:1, :EMB].set(wor).at[1:2, EMB:].set(wor)

    Wd1 = p['dec1_W']
    bd1c = p['dec1_b'].reshape(-1, 1)
    Wd2 = p['dec2_W']
    bd2c = p['dec2_b'].reshape(-1, 1)

    inputs = [x_all, x_allT,
              We1, be1, We2, be2,
              We1T, be1c, We2T, be2c,
              Wo1T, bo1c, Wo2T, bo2c,
              Wq, Wa1q, ba1,
              Wk_r, Wv_r, bv, Whr_rec, Who_rec,
              Wk_sT, Wv_sT,
              Whard1h, Whr_xT, bhrc,
              Whardo1h, bhoc,
              A1k_bd, a2_2, wrr2, crr, wor2, cor,
              Wd1, bd1c, Wd2, bd2c]

    def rep_spec(a):
        nd = a.ndim
        return pl.BlockSpec(a.shape, lambda i, _nd=nd: (0,) * _nd)

    yT = pl.pallas_call(
        _fused,
        grid=(GRID,),
        in_specs=[rep_spec(a) for a in inputs],
        out_specs=pl.BlockSpec((1, 2 * EMB, TR), lambda i: (i, 0, 0)),
        out_shape=jax.ShapeDtypeStruct((GRID, 2 * EMB, TR), f32),
        scratch_shapes=[
            pltpu.VMEM((NR, EMB), f32),
            pltpu.VMEM((NR, EMB), f32),
            pltpu.VMEM((NR, EMB), f32),
            pltpu.VMEM((NR, EMB), f32),
            pltpu.VMEM((NR, EMB), f32),
            pltpu.VMEM((NR, EMB), f32),
            pltpu.VMEM((2 * EMB, NS), f32),
            pltpu.VMEM((2 * EMB, NS), f32),
            pltpu.VMEM((2 * EMB, NR), f32),
            pltpu.VMEM((2 * EMB, NO), f32),
        ],
        compiler_params=pltpu.CompilerParams(
            dimension_semantics=("arbitrary",)),
    )(*inputs)
    return yT.transpose(0, 2, 1).reshape(NR, 2 * EMB)
